# baseline (speedup 1.0000x reference)
import functools

import jax
import jax.numpy as jnp
import numpy as np
from jax import lax
from jax.experimental import pallas as pl
from jax.experimental.pallas import tpu as pltpu
from jax.experimental.pallas import tpu_sc as plsc

B = 16
H_NUM = 3
W_NUM = 9
GRID = 20
DIM = 1024
L = 64
T = H_NUM * W_NUM + 1

NC = 2
NS = 16
NW = NC * NS
AUX_ROWS = 2 * NW
K = 8
ZROW = 2 * GRID + 4


def _aux_index_base() -> np.ndarray:
    gidx = np.full((AUX_ROWS, K), ZROW, dtype=np.int32)
    for t in range(T):
        y = GRID - 1 if t == T - 1 else t // W_NUM
        x = GRID - 1 if t == T - 1 else t % W_NUM
        m = 1 if t == T - 1 else 0
        gidx[t, 0] = y
        gidx[t, 1] = GRID + x
        gidx[t, 2] = 2 * GRID + 2 + m
    return gidx


_AUX_BASE = _aux_index_base()


def _sc_gather_sum(gidx_flat, cat):
    mesh = plsc.VectorSubcoreMesh(core_axis_name="c", subcore_axis_name="s")

    @functools.partial(
        pl.kernel,
        mesh=mesh,
        out_type=jax.ShapeDtypeStruct((AUX_ROWS, DIM), jnp.float32),
        scratch_types=[
            pltpu.VMEM((K,), jnp.int32),
            pltpu.VMEM((K, DIM), jnp.float32),
            pltpu.VMEM((DIM,), jnp.float32),
            pltpu.SemaphoreType.DMA,
        ],
    )
    def sc_fn(gidx_hbm, cat_hbm, aux_hbm, idx_v, rows_v, acc_v, sem):
        wid = lax.axis_index("s") * NC + lax.axis_index("c")
        for j in range(AUX_ROWS // NW):
            r = wid * (AUX_ROWS // NW) + j
            pltpu.sync_copy(gidx_hbm.at[pl.ds(r * K, K)], idx_v)
            pltpu.async_copy(cat_hbm.at[idx_v], rows_v, sem).wait()

            def chunk(i, _):
                s = rows_v[0, pl.ds(i * 16, 16)]
                for k in range(1, K):
                    s = s + rows_v[k, pl.ds(i * 16, 16)]
                acc_v[pl.ds(i * 16, 16)] = s
                return 0

            lax.fori_loop(0, DIM // 16, chunk, 0)
            pltpu.sync_copy(acc_v, aux_hbm.at[r])

    return sc_fn(gidx_flat, cat)


def _tc_body(local_ref, comb_ref, img_ref, out_ref):
    tmp = local_ref[0] + img_ref[0, 0, :][None, :]
    for t in range(T):
        out_ref[0, t * L:(t + 1) * L, :] = tmp + comb_ref[t, 0, :][None, :]


def kernel(image_time, local_pos, row_embed, col_embed, image_embed, type_embed):
    it32 = image_time.astype(jnp.int32)

    cat = jnp.concatenate(
        [row_embed, col_embed, image_embed, type_embed,
         jnp.zeros((1, DIM), jnp.float32)], axis=0)

    gidx = jnp.asarray(_AUX_BASE)
    gidx = gidx.at[T:T + B, 0].set(2 * GRID + it32)
    gidx_flat = gidx.reshape(-1)

    aux = _sc_gather_sum(gidx_flat, cat)
    aux3 = aux.reshape(AUX_ROWS, 1, DIM)

    out = pl.pallas_call(
        _tc_body,
        grid=(B,),
        in_specs=[
            pl.BlockSpec((1, L, DIM), lambda b: (0, 0, 0)),
            pl.BlockSpec((T, 1, DIM), lambda b: (0, 0, 0)),
            pl.BlockSpec((1, 1, DIM), lambda b: (T + b, 0, 0)),
        ],
        out_specs=pl.BlockSpec((1, T * L, DIM), lambda b: (b, 0, 0)),
        out_shape=jax.ShapeDtypeStruct((B, T * L, DIM), jnp.float32),
    )(local_pos, aux3, aux3)
    return out


# speedup vs baseline: 1.2691x; 1.2691x over previous
import functools

import jax
import jax.numpy as jnp
import numpy as np
from jax import lax
from jax.experimental import pallas as pl
from jax.experimental.pallas import tpu as pltpu
from jax.experimental.pallas import tpu_sc as plsc

B = 16
H_NUM = 3
W_NUM = 9
GRID = 20
DIM = 1024
L = 64
T = H_NUM * W_NUM + 1

NC = 2
NS = 16
NW = NC * NS
AUX_ROWS = T + B
K = 4
ZROW = 2 * GRID + 4


def _aux_index_base() -> np.ndarray:
    gidx = np.full((AUX_ROWS, K), ZROW, dtype=np.int32)
    for t in range(T):
        y = GRID - 1 if t == T - 1 else t // W_NUM
        x = GRID - 1 if t == T - 1 else t % W_NUM
        m = 1 if t == T - 1 else 0
        gidx[t, 0] = y
        gidx[t, 1] = GRID + x
        gidx[t, 2] = 2 * GRID + 2 + m
    return gidx


_AUX_BASE = _aux_index_base()


def _sc_gather_sum(gidx, cat):
    mesh = plsc.VectorSubcoreMesh(core_axis_name="c", subcore_axis_name="s")

    @functools.partial(
        pl.kernel,
        mesh=mesh,
        out_type=jax.ShapeDtypeStruct((AUX_ROWS, DIM), jnp.float32),
        scratch_types=[
            pltpu.VMEM((K,), jnp.int32),
            pltpu.VMEM((K,), jnp.int32),
            pltpu.VMEM((K, DIM), jnp.float32),
            pltpu.VMEM((K, DIM), jnp.float32),
            pltpu.VMEM((DIM,), jnp.float32),
            pltpu.SemaphoreType.DMA,
            pltpu.SemaphoreType.DMA,
        ],
    )
    def sc_fn(gidx_hbm, cat_hbm, aux_hbm, idx0_v, idx1_v, rows0_v, rows1_v,
              acc_v, sem0, sem1):
        wid = lax.axis_index("s") * NC + lax.axis_index("c")
        has2 = wid < AUX_ROWS - NW
        r1 = jnp.where(has2, NW + wid, wid)

        pltpu.sync_copy(gidx_hbm.at[wid], idx0_v)
        g0 = pltpu.async_copy(cat_hbm.at[idx0_v], rows0_v, sem0)
        pltpu.sync_copy(gidx_hbm.at[r1], idx1_v)
        g1 = pltpu.async_copy(cat_hbm.at[idx1_v], rows1_v, sem1)

        def reduce_to(rows_v, dst_r):
            for i in range(DIM // 16):
                s = rows_v[0, pl.ds(i * 16, 16)]
                for k in range(1, K):
                    s = s + rows_v[k, pl.ds(i * 16, 16)]
                acc_v[pl.ds(i * 16, 16)] = s
            pltpu.sync_copy(acc_v, aux_hbm.at[dst_r])

        g0.wait()
        reduce_to(rows0_v, wid)
        g1.wait()

        @pl.when(has2)
        def _():
            reduce_to(rows1_v, r1)

    return sc_fn(gidx, cat)


def _tc_body(local_ref, comb_ref, img_ref, out_ref):
    tmp = local_ref[0] + img_ref[0, 0, :][None, :]
    for t in range(T):
        out_ref[0, t * L:(t + 1) * L, :] = tmp + comb_ref[t, 0, :][None, :]


def kernel(image_time, local_pos, row_embed, col_embed, image_embed, type_embed):
    it32 = image_time.astype(jnp.int32)

    cat = jnp.concatenate(
        [row_embed, col_embed, image_embed, type_embed,
         jnp.zeros((1, DIM), jnp.float32)], axis=0)

    gidx = jnp.asarray(_AUX_BASE)
    gidx = gidx.at[T:T + B, 0].set(2 * GRID + it32)

    aux = _sc_gather_sum(gidx, cat)
    aux3 = aux.reshape(AUX_ROWS, 1, DIM)

    out = pl.pallas_call(
        _tc_body,
        grid=(B,),
        in_specs=[
            pl.BlockSpec((1, L, DIM), lambda b: (0, 0, 0)),
            pl.BlockSpec((T, 1, DIM), lambda b: (0, 0, 0)),
            pl.BlockSpec((1, 1, DIM), lambda b: (T + b, 0, 0)),
        ],
        out_specs=pl.BlockSpec((1, T * L, DIM), lambda b: (b, 0, 0)),
        out_shape=jax.ShapeDtypeStruct((B, T * L, DIM), jnp.float32),
    )(local_pos, aux3, aux3)
    return out


# speedup vs baseline: 1.4865x; 1.1712x over previous
import functools

import jax
import jax.numpy as jnp
from jax import lax
from jax.experimental import pallas as pl
from jax.experimental.pallas import tpu as pltpu
from jax.experimental.pallas import tpu_sc as plsc

B = 16
H_NUM = 3
W_NUM = 9
GRID = 20
DIM = 1024
L = 64
T = H_NUM * W_NUM + 1

NC = 2
NS = 16


def _sc_comb(row_embed, col_embed, type_embed):
    mesh = plsc.VectorSubcoreMesh(core_axis_name="c", subcore_axis_name="s")

    @functools.partial(
        pl.kernel,
        mesh=mesh,
        out_type=jax.ShapeDtypeStruct((T, DIM), jnp.float32),
        scratch_types=[
            pltpu.VMEM((DIM,), jnp.float32),
            pltpu.VMEM((DIM,), jnp.float32),
            pltpu.VMEM((DIM,), jnp.float32),
            pltpu.VMEM((DIM,), jnp.float32),
            pltpu.SemaphoreType.DMA,
            pltpu.SemaphoreType.DMA,
            pltpu.SemaphoreType.DMA,
        ],
    )
    def sc_fn(row_hbm, col_hbm, typ_hbm, comb_hbm, ry_v, rx_v, rm_v, acc_v,
              sem0, sem1, sem2):
        wid = lax.axis_index("s") * NC + lax.axis_index("c")
        r = jnp.minimum(wid, T - 1)
        is_thumb = r == T - 1
        y = jnp.where(is_thumb, GRID - 1, r // W_NUM)
        x = jnp.where(is_thumb, GRID - 1, r % W_NUM)
        m = jnp.where(is_thumb, 1, 0)
        g0 = pltpu.async_copy(row_hbm.at[y], ry_v, sem0)
        g1 = pltpu.async_copy(col_hbm.at[x], rx_v, sem1)
        g2 = pltpu.async_copy(typ_hbm.at[m], rm_v, sem2)
        g0.wait()
        g1.wait()
        g2.wait()
        for i in range(DIM // 16):
            d = pl.ds(i * 16, 16)
            acc_v[d] = ry_v[d] + rx_v[d] + rm_v[d]

        @pl.when(wid < T)
        def _():
            pltpu.sync_copy(acc_v, comb_hbm.at[wid])

    return sc_fn(row_embed, col_embed, type_embed)


def _tc_body(it_ref, local_ref, comb_ref, img_ref, out_ref):
    b = pl.program_id(0)
    it = it_ref[b]
    img = jnp.where(it == 0, img_ref[0, :], img_ref[1, :])
    tmp = local_ref[0] + img[None, :]
    for t in range(T):
        out_ref[0, t * L:(t + 1) * L, :] = tmp + comb_ref[t, :][None, :]


def kernel(image_time, local_pos, row_embed, col_embed, image_embed, type_embed):
    it32 = image_time.astype(jnp.int32)
    comb = _sc_comb(row_embed, col_embed, type_embed)

    out = pl.pallas_call(
        _tc_body,
        grid=(B,),
        in_specs=[
            pl.BlockSpec(memory_space=pltpu.SMEM),
            pl.BlockSpec((1, L, DIM), lambda b: (0, 0, 0)),
            pl.BlockSpec((T, DIM), lambda b: (0, 0)),
            pl.BlockSpec((2, DIM), lambda b: (0, 0)),
        ],
        out_specs=pl.BlockSpec((1, T * L, DIM), lambda b: (b, 0, 0)),
        out_shape=jax.ShapeDtypeStruct((B, T * L, DIM), jnp.float32),
    )(it32, local_pos, comb, image_embed)
    return out


# speedup vs baseline: 1.4936x; 1.0048x over previous
import functools

import jax
import jax.numpy as jnp
from jax import lax
from jax.experimental import pallas as pl
from jax.experimental.pallas import tpu as pltpu
from jax.experimental.pallas import tpu_sc as plsc

B = 16
H_NUM = 3
W_NUM = 9
GRID = 20
DIM = 1024
L = 64
T = H_NUM * W_NUM + 1

NC = 2
NS = 16


def _sc_comb(row_embed, col_embed, type_embed):
    mesh = plsc.VectorSubcoreMesh(core_axis_name="c", subcore_axis_name="s")

    @functools.partial(
        pl.kernel,
        mesh=mesh,
        out_type=jax.ShapeDtypeStruct((T, DIM), jnp.float32),
        scratch_types=[
            pltpu.VMEM((DIM,), jnp.float32),
            pltpu.VMEM((DIM,), jnp.float32),
            pltpu.VMEM((DIM,), jnp.float32),
            pltpu.VMEM((DIM,), jnp.float32),
            pltpu.SemaphoreType.DMA,
            pltpu.SemaphoreType.DMA,
            pltpu.SemaphoreType.DMA,
        ],
    )
    def sc_fn(row_hbm, col_hbm, typ_hbm, comb_hbm, ry_v, rx_v, rm_v, acc_v,
              sem0, sem1, sem2):
        wid = lax.axis_index("s") * NC + lax.axis_index("c")
        r = jnp.minimum(wid, T - 1)
        is_thumb = r == T - 1
        y = jnp.where(is_thumb, GRID - 1, r // W_NUM)
        x = jnp.where(is_thumb, GRID - 1, r % W_NUM)
        m = jnp.where(is_thumb, 1, 0)
        g0 = pltpu.async_copy(row_hbm.at[y], ry_v, sem0)
        g1 = pltpu.async_copy(col_hbm.at[x], rx_v, sem1)
        g2 = pltpu.async_copy(typ_hbm.at[m], rm_v, sem2)
        g0.wait()
        g1.wait()
        g2.wait()

        UNROLL = 4

        def chunk(j, _):
            for u in range(UNROLL):
                d = pl.ds((j * UNROLL + u) * 16, 16)
                acc_v[d] = ry_v[d] + rx_v[d] + rm_v[d]
            return 0

        lax.fori_loop(0, DIM // 16 // UNROLL, chunk, 0)

        @pl.when(wid < T)
        def _():
            pltpu.sync_copy(acc_v, comb_hbm.at[wid])

    return sc_fn(row_embed, col_embed, type_embed)


def _tc_body(it_ref, local_ref, comb_ref, img_ref, out_ref):
    b = pl.program_id(0)
    it = it_ref[b]
    img = jnp.where(it == 0, img_ref[0, :], img_ref[1, :])
    tmp = local_ref[0] + img[None, :]
    for t in range(T):
        out_ref[0, t * L:(t + 1) * L, :] = tmp + comb_ref[t, :][None, :]


def kernel(image_time, local_pos, row_embed, col_embed, image_embed, type_embed):
    it32 = image_time.astype(jnp.int32)
    comb = _sc_comb(row_embed, col_embed, type_embed)

    out = pl.pallas_call(
        _tc_body,
        grid=(B,),
        in_specs=[
            pl.BlockSpec(memory_space=pltpu.SMEM),
            pl.BlockSpec((1, L, DIM), lambda b: (0, 0, 0)),
            pl.BlockSpec((T, DIM), lambda b: (0, 0)),
            pl.BlockSpec((2, DIM), lambda b: (0, 0)),
        ],
        out_specs=pl.BlockSpec((1, T * L, DIM), lambda b: (b, 0, 0)),
        out_shape=jax.ShapeDtypeStruct((B, T * L, DIM), jnp.float32),
    )(it32, local_pos, comb, image_embed)
    return out


# speedup vs baseline: 1.4999x; 1.0043x over previous
import functools

import jax
import jax.numpy as jnp
from jax import lax
from jax.experimental import pallas as pl
from jax.experimental.pallas import tpu as pltpu
from jax.experimental.pallas import tpu_sc as plsc

B = 16
H_NUM = 3
W_NUM = 9
GRID = 20
DIM = 1024
L = 64
T = H_NUM * W_NUM + 1

NC = 2
NS = 16


def _sc_comb(row_embed, col_embed, type_embed):
    mesh = plsc.VectorSubcoreMesh(core_axis_name="c", subcore_axis_name="s",
                                  num_cores=1)

    @functools.partial(
        pl.kernel,
        mesh=mesh,
        out_type=jax.ShapeDtypeStruct((T, DIM), jnp.float32),
        scratch_types=[
            pltpu.VMEM((2, DIM), jnp.float32),
            pltpu.VMEM((2, DIM), jnp.float32),
            pltpu.VMEM((2, DIM), jnp.float32),
            pltpu.VMEM((DIM,), jnp.float32),
            pltpu.SemaphoreType.DMA,
            pltpu.SemaphoreType.DMA,
            pltpu.SemaphoreType.DMA,
            pltpu.SemaphoreType.DMA,
            pltpu.SemaphoreType.DMA,
            pltpu.SemaphoreType.DMA,
        ],
    )
    def sc_fn(row_hbm, col_hbm, typ_hbm, comb_hbm, ry_v, rx_v, rm_v, acc_v,
              s0a, s1a, s2a, s0b, s1b, s2b):
        wid = lax.axis_index("s")

        def idxs(r):
            is_thumb = r == T - 1
            y = jnp.where(is_thumb, GRID - 1, r // W_NUM)
            x = jnp.where(is_thumb, GRID - 1, r % W_NUM)
            m = jnp.where(is_thumb, 1, 0)
            return y, x, m

        r0 = wid
        r1 = jnp.minimum(wid + NS, T - 1)
        y0, x0, m0 = idxs(r0)
        y1, x1, m1 = idxs(r1)
        ga = (pltpu.async_copy(row_hbm.at[y0], ry_v.at[0], s0a),
              pltpu.async_copy(col_hbm.at[x0], rx_v.at[0], s1a),
              pltpu.async_copy(typ_hbm.at[m0], rm_v.at[0], s2a))
        gb = (pltpu.async_copy(row_hbm.at[y1], ry_v.at[1], s0b),
              pltpu.async_copy(col_hbm.at[x1], rx_v.at[1], s1b),
              pltpu.async_copy(typ_hbm.at[m1], rm_v.at[1], s2b))

        UNROLL = 4

        def reduce_to(k, dst_r):
            def chunk(j, _):
                for u in range(UNROLL):
                    d = pl.ds((j * UNROLL + u) * 16, 16)
                    acc_v[d] = ry_v[k, d] + rx_v[k, d] + rm_v[k, d]
                return 0

            lax.fori_loop(0, DIM // 16 // UNROLL, chunk, 0)
            pltpu.sync_copy(acc_v, comb_hbm.at[dst_r])

        for g in ga:
            g.wait()
        reduce_to(0, r0)
        for g in gb:
            g.wait()

        @pl.when(wid < T - NS)
        def _():
            reduce_to(1, wid + NS)

    return sc_fn(row_embed, col_embed, type_embed)


def _tc_body(it_ref, local_ref, comb_ref, img_ref, out_ref):
    b = pl.program_id(0)
    it = it_ref[b]
    img = jnp.where(it == 0, img_ref[0, :], img_ref[1, :])
    tmp = local_ref[0] + img[None, :]
    for t in range(T):
        out_ref[0, t * L:(t + 1) * L, :] = tmp + comb_ref[t, :][None, :]


def kernel(image_time, local_pos, row_embed, col_embed, image_embed, type_embed):
    it32 = image_time.astype(jnp.int32)
    comb = _sc_comb(row_embed, col_embed, type_embed)

    out = pl.pallas_call(
        _tc_body,
        grid=(B,),
        in_specs=[
            pl.BlockSpec(memory_space=pltpu.SMEM),
            pl.BlockSpec((1, L, DIM), lambda b: (0, 0, 0)),
            pl.BlockSpec((T, DIM), lambda b: (0, 0)),
            pl.BlockSpec((2, DIM), lambda b: (0, 0)),
        ],
        out_specs=pl.BlockSpec((1, T * L, DIM), lambda b: (b, 0, 0)),
        out_shape=jax.ShapeDtypeStruct((B, T * L, DIM), jnp.float32),
    )(it32, local_pos, comb, image_embed)
    return out
